# Initial kernel scaffold; baseline (speedup 1.0000x reference)
import jax
import jax.numpy as jnp
from jax.experimental import pallas as pl


def kernel(x, edge_attr, edge_index, batch, W1, att_src1, att_dst1, b1, W2, att_src2, att_dst2, b2, Wc, bc):
    raise NotImplementedError("write your pallas kernel here")


# speedup vs baseline: 9.1517x; 9.1517x over previous
import functools

import jax
import jax.numpy as jnp
from jax import lax
from jax.experimental import pallas as pl
from jax.experimental.pallas import tpu as pltpu
from jax.experimental.pallas import tpu_sc as plsc

N = 10000
E = 320000
D = 128
NG = 64
N_OUT = 64

NPAD = 10240
ROWS_PER_TILE = NPAD // 16
CH1 = 157 * 128
EPAD = 16 * CH1
CH2 = CH1 // 2
STEPS2 = CH2 // 16


def _mm_aux_body(x_ref, w_ref, wa_ref, xpa_ref, xpb_ref, aux_ref):
    xp = jnp.dot(x_ref[...], w_ref[...], preferred_element_type=jnp.float32)
    xpa_ref[...] = xp[:, :D // 2]
    xpb_ref[...] = xp[:, D // 2:]
    aux_ref[...] = jnp.dot(xp, wa_ref[...], preferred_element_type=jnp.float32)


def _mm_aux(x, w, wa, blk):
    n = x.shape[0]
    grid = n // blk
    return pl.pallas_call(
        _mm_aux_body,
        grid=(grid,),
        in_specs=[
            pl.BlockSpec((blk, D), lambda i: (i, 0)),
            pl.BlockSpec((D, D), lambda i: (0, 0)),
            pl.BlockSpec((D, D), lambda i: (0, 0)),
        ],
        out_specs=[
            pl.BlockSpec((blk, D // 2), lambda i: (i, 0)),
            pl.BlockSpec((blk, D // 2), lambda i: (i, 0)),
            pl.BlockSpec((blk, D), lambda i: (i, 0)),
        ],
        out_shape=[
            jax.ShapeDtypeStruct((n, D // 2), jnp.float32),
            jax.ShapeDtypeStruct((n, D // 2), jnp.float32),
            jax.ShapeDtypeStruct((n, D), jnp.float32),
        ],
    )(x, w, wa)


def _combine_halves(p_ref, b_ref):
    ha = p_ref[0, 0] + p_ref[1, 0]
    hb = p_ref[0, 1] + p_ref[1, 1]
    h = jnp.concatenate([ha, hb], axis=-1)
    return jnp.maximum(h + b_ref[...], 0.0)


def _combine_mm_aux_body(p_ref, b_ref, w_ref, wa_ref, xpa_ref, xpb_ref,
                         aux_ref):
    h = _combine_halves(p_ref, b_ref)
    xp = jnp.dot(h, w_ref[...], preferred_element_type=jnp.float32)
    xpa_ref[...] = xp[:, :D // 2]
    xpb_ref[...] = xp[:, D // 2:]
    aux_ref[...] = jnp.dot(xp, wa_ref[...], preferred_element_type=jnp.float32)


def _combine_mm_aux(p, b, w, wa, blk):
    n = p.shape[2]
    grid = n // blk
    return pl.pallas_call(
        _combine_mm_aux_body,
        grid=(grid,),
        in_specs=[
            pl.BlockSpec((2, 2, blk, D // 2), lambda i: (0, 0, i, 0)),
            pl.BlockSpec((1, D), lambda i: (0, 0)),
            pl.BlockSpec((D, D), lambda i: (0, 0)),
            pl.BlockSpec((D, D), lambda i: (0, 0)),
        ],
        out_specs=[
            pl.BlockSpec((blk, D // 2), lambda i: (i, 0)),
            pl.BlockSpec((blk, D // 2), lambda i: (i, 0)),
            pl.BlockSpec((blk, D), lambda i: (i, 0)),
        ],
        out_shape=[
            jax.ShapeDtypeStruct((n, D // 2), jnp.float32),
            jax.ShapeDtypeStruct((n, D // 2), jnp.float32),
            jax.ShapeDtypeStruct((n, D), jnp.float32),
        ],
    )(p, b, w, wa)


def _pool_body(p_ref, b_ref, batch_ref, wc_ref, bc_ref, out_ref, acc_ref):
    i = pl.program_id(0)
    ng = pl.num_programs(0)

    @pl.when(i == 0)
    def _():
        acc_ref[...] = jnp.zeros_like(acc_ref)

    h = _combine_halves(p_ref, b_ref)
    seg = batch_ref[...][0]
    gids = lax.broadcasted_iota(jnp.int32, (NG, seg.shape[1]), 0)
    onehot = (gids == seg).astype(jnp.float32)
    acc_ref[...] += jnp.dot(onehot, h, preferred_element_type=jnp.float32)

    @pl.when(i == ng - 1)
    def _():
        out_ref[...] = (
            jnp.dot(acc_ref[...], wc_ref[...],
                    preferred_element_type=jnp.float32)
            + bc_ref[...]
        )


def _pool_classify(p, b, batch, wc, bc, blk):
    n = p.shape[2]
    grid = n // blk
    batch3d = batch.reshape(grid, 1, blk)
    return pl.pallas_call(
        _pool_body,
        grid=(grid,),
        in_specs=[
            pl.BlockSpec((2, 2, blk, D // 2), lambda i: (0, 0, i, 0)),
            pl.BlockSpec((1, D), lambda i: (0, 0)),
            pl.BlockSpec((1, 1, blk), lambda i: (i, 0, 0)),
            pl.BlockSpec((D, N_OUT), lambda i: (0, 0)),
            pl.BlockSpec((1, N_OUT), lambda i: (0, 0)),
        ],
        out_specs=pl.BlockSpec((NG, N_OUT), lambda i: (0, 0)),
        out_shape=jax.ShapeDtypeStruct((NG, N_OUT), jnp.float32),
        scratch_shapes=[pltpu.VMEM((NG, D), jnp.float32)],
    )(p, b, batch3d, wc, bc)


DH = D // 2


def _gat_sc_layer(xp_a, xp_b, a_src, a_dst, src_p, dst_p):
    mesh = plsc.VectorSubcoreMesh(core_axis_name="c", subcore_axis_name="s",
                                  num_cores=2, num_subcores=16)

    @functools.partial(
        pl.kernel,
        out_type=jax.ShapeDtypeStruct((2, 2, NPAD, DH), jnp.float32),
        mesh=mesh,
        compiler_params=pltpu.CompilerParams(
            use_tc_tiling_on_sc=False, needs_layout_passes=False),
        scratch_types=[
            pltpu.VMEM((N,), jnp.float32),
            pltpu.VMEM((N,), jnp.float32),
            pltpu.VMEM((CH1,), jnp.int32),
            pltpu.VMEM((CH1,), jnp.int32),
            pltpu.VMEM((128,), jnp.int32),
            pltpu.VMEM((128,), jnp.float32),
            pltpu.VMEM((NPAD,), jnp.float32),
            pltpu.VMEM((16, DH), jnp.float32),
            pltpu.VMEM((64, DH), jnp.float32),
            pltpu.VMEM((ROWS_PER_TILE,), jnp.float32),
            pltpu.VMEM_SHARED((NPAD,), jnp.float32),
            pltpu.VMEM_SHARED((NPAD, DH), jnp.float32),
            pltpu.SemaphoreType.DMA,
        ],
    )
    def body(xpa_hbm, xpb_hbm, asrc_hbm, adst_hbm, src_hbm, dst_hbm, out_hbm,
             asrc_v, adst_v, src_v, dst_v, dstrow_v, exbuf_v, denom_v, rows_v,
             zbuf_v, zden_v, denom_sh, out_sh, sem):
        c = lax.axis_index("c")
        s = lax.axis_index("s")

        pltpu.sync_copy(asrc_hbm, asrc_v)
        pltpu.sync_copy(adst_hbm, adst_v)
        base1 = s * CH1
        pltpu.sync_copy(src_hbm.at[pl.ds(base1, CH1)], src_v)
        pltpu.sync_copy(dst_hbm.at[pl.ds(base1, CH1)], dst_v)

        for i in range(64):
            for k in range(DH // 16):
                zbuf_v[i, pl.ds(k * 16, 16)] = jnp.zeros((16,), jnp.float32)
        for i in range(ROWS_PER_TILE // 16):
            zden_v[pl.ds(i * 16, 16)] = jnp.zeros((16,), jnp.float32)
        rbase = s * ROWS_PER_TILE
        pltpu.sync_copy(zden_v, denom_sh.at[pl.ds(rbase, ROWS_PER_TILE)])
        plsc.subcore_barrier()

        def alpha_ex(off):
            isrc = src_v[pl.ds(off, 16)]
            idst = dst_v[pl.ds(off, 16)]
            av = plsc.load_gather(asrc_v, [isrc])
            bv = plsc.load_gather(adst_v, [idst])
            al = av + bv
            al = jnp.where(al >= 0.0, al, 0.2 * al)
            exv = jnp.exp(al)
            ge = base1 + off + lax.broadcasted_iota(jnp.int32, (16,), 0)
            exv = jnp.where(ge < E, exv, 0.0)
            return isrc, idst, exv

        def p1(j, carry):
            pltpu.sync_copy(dst_hbm.at[pl.ds(base1 + j * 128, 128)], dstrow_v)
            for k in range(8):
                _, _, exv = alpha_ex(j * 128 + k * 16)
                exbuf_v[pl.ds(k * 16, 16)] = exv
            pltpu.sync_copy(exbuf_v, denom_sh.at[dstrow_v], add=True)
            return carry

        lax.fori_loop(0, 157, p1, 0)
        plsc.subcore_barrier()
        pltpu.sync_copy(denom_sh, denom_v)

        local0 = c * CH2

        def run_half(xph_hbm, half):
            for r in range(ROWS_PER_TILE // 64):
                pltpu.sync_copy(zbuf_v, out_sh.at[pl.ds(rbase + r * 64, 64), :])
            plsc.subcore_barrier()

            def p2(i, carry):
                off = local0 + i * 16
                isrc, idst, exv = alpha_ex(off)
                dn = plsc.load_gather(denom_v, [idst])
                coef = exv / (dn + 1e-16)
                cp = pltpu.async_copy(xph_hbm.at[isrc], rows_v, sem)
                cp.wait()
                for e in range(16):
                    ce = coef[e]
                    for k in range(DH // 16):
                        sl = pl.ds(k * 16, 16)
                        rows_v[e, sl] = rows_v[e, sl] * ce
                pltpu.sync_copy(rows_v, out_sh.at[idst], add=True)
                return carry

            lax.fori_loop(0, STEPS2, p2, 0)
            plsc.subcore_barrier()
            pltpu.sync_copy(
                out_sh.at[pl.ds(rbase, ROWS_PER_TILE), :],
                out_hbm.at[c, half, pl.ds(rbase, ROWS_PER_TILE), :])
            plsc.subcore_barrier()

        run_half(xpa_hbm, 0)
        run_half(xpb_hbm, 1)

    return body(xp_a, xp_b, a_src, a_dst, src_p, dst_p)


def _att_matrix(att_src, att_dst):
    wa = jnp.zeros((D, D), jnp.float32)
    wa = wa.at[:, 0].set(att_src[0])
    wa = wa.at[:, 1].set(att_dst[0])
    return wa


def kernel(x, edge_attr, edge_index, batch, W1, att_src1, att_dst1, b1,
           W2, att_src2, att_dst2, b2, Wc, bc):
    del edge_attr
    src = edge_index[0]
    dst = edge_index[1]
    pad = EPAD - E
    src_p = jnp.concatenate([src, jnp.zeros((pad,), jnp.int32)])
    dst_p = jnp.concatenate([dst, jnp.zeros((pad,), jnp.int32)])

    wa1 = _att_matrix(att_src1, att_dst1)
    wa2 = _att_matrix(att_src2, att_dst2)

    xp1a, xp1b, aux1 = _mm_aux(x, W1, wa1, blk=1000)
    p1 = _gat_sc_layer(xp1a, xp1b, aux1[:, 0], aux1[:, 1], src_p, dst_p)

    xp2a, xp2b, aux2 = _combine_mm_aux(p1, b1.reshape(1, D), W2, wa2,
                                       blk=1024)
    p2 = _gat_sc_layer(xp2a, xp2b, aux2[:N, 0], aux2[:N, 1], src_p, dst_p)

    batch_pad = jnp.concatenate(
        [batch, jnp.full((NPAD - N,), NG, jnp.int32)])
    out = _pool_classify(p2, b2.reshape(1, D), batch_pad,
                         Wc, bc.reshape(1, N_OUT), blk=1024)
    return out
